# baseline (speedup 1.0000x reference)
import jax
import jax.numpy as jnp
from jax import lax
from jax.experimental import pallas as pl
from jax.experimental.pallas import tpu as pltpu

_W = 64


def _roll_up(z, sh):
    return jnp.concatenate([z[sh:], z[:sh]], axis=0)


def _shift_by_lane(z, amt_masks):
    for b, m in enumerate(amt_masks):
        sh = 1 << b
        z = jnp.where(m, _roll_up(z, sh), z)
    return z


def _cmpex_small(w, row, k, j):
    up = _roll_up(w, j)
    dn = _roll_up(w, _W - j)
    bitj = (row & j) == 0
    p = jnp.where(bitj, up, dn)
    tm = bitj if k == _W else ((row & k) == 0) == bitj
    return jnp.where(tm, jnp.minimum(w, p), jnp.maximum(w, p))


def _cmpex_large(w, k, j):
    C = w.shape[1]
    if k == _W:
        M = _W // (2 * j)
        wv = w.reshape(M, 2, j, C)
        a, b = wv[:, 0], wv[:, 1]
        mn, mx = jnp.minimum(a, b), jnp.maximum(a, b)
        return jnp.concatenate([mn[:, None], mx[:, None]], axis=1).reshape(_W, C)
    G = _W // (2 * k)
    M = k // (2 * j)
    wv = w.reshape(G, 2, M, 2, j, C)
    a, b = wv[:, :, :, 0], wv[:, :, :, 1]
    mn, mx = jnp.minimum(a, b), jnp.maximum(a, b)
    h0 = jnp.concatenate([mn[:, 0:1], mx[:, 1:2]], axis=1)
    h1 = jnp.concatenate([mx[:, 0:1], mn[:, 1:2]], axis=1)
    return jnp.concatenate(
        [h0[:, :, :, None], h1[:, :, :, None]], axis=3).reshape(_W, C)


def _sort64(w, row):
    k = 2
    while k <= _W:
        j = k // 2
        while j > 0:
            if j >= 8:
                w = _cmpex_large(w, k, j)
            else:
                w = _cmpex_small(w, row, k, j)
            j //= 2
        k *= 2
    return w


def _windowed_sort_kernel(v_ref, o_ref):
    L, C = v_ref.shape[1], v_ref.shape[2]
    n_strips = L // _W
    lane = lax.broadcasted_iota(jnp.int32, (1, C), 1) & (_W - 1)
    row = lax.broadcasted_iota(jnp.int32, (_W, 1), 0)
    fwd_masks = [(lane & (1 << b)) != 0 for b in range(6)]
    amt = _W - lane
    inv_masks = [(amt & (1 << b)) != 0 for b in range(7)]

    def sort_strip(z):
        return _sort64(_shift_by_lane(z, fwd_masks)[:_W], row)

    x_head = v_ref[0, : _W, :]
    x_tail = v_ref[0, L - _W :, :]
    w_last = sort_strip(jnp.concatenate([x_tail, x_head], axis=0))

    def body(s, w_prev):
        w_s = sort_strip(v_ref[0, pl.ds(_W * s, 2 * _W), :])
        z2 = jnp.concatenate([w_prev, w_s], axis=0)
        o_ref[0, pl.ds(_W * s, _W), :] = _shift_by_lane(z2, inv_masks)[:_W]
        return w_s

    w_prev = lax.fori_loop(0, n_strips - 1, body, w_last)
    z2 = jnp.concatenate([w_prev, w_last], axis=0)
    o_ref[0, L - _W :, :] = _shift_by_lane(z2, inv_masks)[:_W]


def kernel(q, k, v):
    B, L, D = v.shape
    C = 256
    grid = (B, D // C)
    return pl.pallas_call(
        _windowed_sort_kernel,
        grid=grid,
        in_specs=[pl.BlockSpec((1, L, C), lambda b, c: (b, 0, c))],
        out_specs=pl.BlockSpec((1, L, C), lambda b, c: (b, 0, c)),
        out_shape=jax.ShapeDtypeStruct(v.shape, v.dtype),
        compiler_params=pltpu.CompilerParams(
            dimension_semantics=("parallel", "parallel"),
        ),
    )(v)
